# Initial kernel scaffold; baseline (speedup 1.0000x reference)
import functools

import jax
import jax.numpy as jnp
from jax.experimental import pallas as pl
from jax.experimental.pallas import tpu as pltpu

N_GRAPH = 64


def _body(x_ref, idx_ref, pW1_ref, pb1_ref, pW2_ref, pb2_ref,
          vW1_ref, vb1_ref, vW2_ref, vb2_ref, vW3_ref, vb3_ref,
          spin_ref, values_ref, counts_ref, emb0_ref):
    i = pl.program_id(0)
    nb = pl.num_programs(0)

    xb = x_ref[...]
    h = jnp.dot(xb, pW1_ref[...], preferred_element_type=jnp.float32)
    h = jnp.maximum(h + pb1_ref[...], 0.0)
    s = jnp.dot(h, pW2_ref[...], preferred_element_type=jnp.float32)
    spin_ref[...] = jnp.maximum(s + pb2_ref[...], 0.0)

    idx = idx_ref[0, 0, :]
    bsz = idx.shape[0]
    gids = jax.lax.broadcasted_iota(jnp.int32, (bsz, N_GRAPH), 1)
    onehot = (idx[:, None] == gids).astype(jnp.float32)
    ones_row = jnp.ones((1, bsz), jnp.float32)
    blk_counts = jnp.dot(ones_row, onehot,
                         preferred_element_type=jnp.float32)
    mask0 = (idx == 0).astype(jnp.float32)[None, :]
    blk_emb0 = jnp.dot(mask0, xb,
                       preferred_element_type=jnp.float32)

    @pl.when(i == 0)
    def _init():
        counts_ref[...] = blk_counts
        emb0_ref[...] = blk_emb0

    @pl.when(i != 0)
    def _acc():
        counts_ref[...] += blk_counts
        emb0_ref[...] += blk_emb0

    @pl.when(i == nb - 1)
    def _final():
        n = counts_ref[...]
        scale = jnp.sqrt(n).reshape(N_GRAPH, 1)
        z = emb0_ref[...] / scale
        h1 = jnp.dot(z, vW1_ref[...], preferred_element_type=jnp.float32)
        h1 = jnp.maximum(h1 + vb1_ref[...], 0.0)
        h2 = jnp.dot(h1, vW2_ref[...], preferred_element_type=jnp.float32)
        h2 = jnp.maximum(h2 + vb2_ref[...], 0.0)
        h3 = jnp.dot(h2, vW3_ref[...], preferred_element_type=jnp.float32)
        values_ref[...] = jnp.maximum(h3 + vb3_ref[...], 0.0)


@functools.partial(jax.jit, static_argnames=("interpret",))
def _run(x, idx, pW1, pb1, pW2, pb2, vW1, vb1, vW2, vb2, vW3, vb3,
         interpret=False):
    n_nodes, d_feat = x.shape
    block = 5000
    if n_nodes % block != 0:
        block = 8
        while n_nodes % (block * 2) == 0 and block < 4096:
            block *= 2
    nb = n_nodes // block

    idx3 = idx.reshape(nb, 1, block)

    grid_spec = pl.GridSpec(
        grid=(nb,),
        in_specs=[
            pl.BlockSpec((block, d_feat), lambda i: (i, 0)),
            pl.BlockSpec((1, 1, block), lambda i: (i, 0, 0)),
            pl.BlockSpec(pW1.shape, lambda i: (0, 0)),
            pl.BlockSpec((1, pb1.shape[0]), lambda i: (0, 0)),
            pl.BlockSpec(pW2.shape, lambda i: (0, 0)),
            pl.BlockSpec((1, pb2.shape[0]), lambda i: (0, 0)),
            pl.BlockSpec(vW1.shape, lambda i: (0, 0)),
            pl.BlockSpec((1, vb1.shape[0]), lambda i: (0, 0)),
            pl.BlockSpec(vW2.shape, lambda i: (0, 0)),
            pl.BlockSpec((1, vb2.shape[0]), lambda i: (0, 0)),
            pl.BlockSpec(vW3.shape, lambda i: (0, 0)),
            pl.BlockSpec((1, vb3.shape[0]), lambda i: (0, 0)),
        ],
        out_specs=[
            pl.BlockSpec((block, 2), lambda i: (i, 0)),
            pl.BlockSpec((N_GRAPH, 1), lambda i: (0, 0)),
        ],
    )

    spin, values = pl.pallas_call(
        _body,
        grid_spec=grid_spec,
        out_shape=[
            jax.ShapeDtypeStruct((n_nodes, 2), jnp.float32),
            jax.ShapeDtypeStruct((N_GRAPH, 1), jnp.float32),
        ],
        scratch_shapes=[
            pltpu.VMEM((1, N_GRAPH), jnp.float32),
            pltpu.VMEM((1, d_feat), jnp.float32),
        ],
        interpret=interpret,
    )(x, idx3, pW1, pb1.reshape(1, -1), pW2, pb2.reshape(1, -1),
      vW1, vb1.reshape(1, -1), vW2, vb2.reshape(1, -1),
      vW3, vb3.reshape(1, -1))
    return spin, values[:, 0]


def kernel(x, node_graph_idx, pW1, pb1, pW2, pb2, vW1, vb1, vW2, vb2, vW3, vb3):
    idx = node_graph_idx.astype(jnp.int32)
    return _run(x, idx, pW1, pb1, pW2, pb2, vW1, vb1, vW2, vb2, vW3, vb3)


# speedup vs baseline: 4.3622x; 4.3622x over previous
import functools

import jax
import jax.numpy as jnp
from jax.experimental import pallas as pl
from jax.experimental.pallas import tpu as pltpu

N_GRAPH = 64


def _body(x_ref, idx_ref, pW1_ref, pb1_ref, pW2_ref, pb2_ref,
          vW1_ref, vb1_ref, vW2_ref, vb2_ref, vW3_ref, vb3_ref,
          spin_ref, values_ref, counts_ref, emb0_ref):
    i = pl.program_id(0)
    nb = pl.num_programs(0)

    xb = x_ref[...]
    h = jnp.dot(xb, pW1_ref[...], preferred_element_type=jnp.float32)
    h = jnp.maximum(h + pb1_ref[...], 0.0)
    s = jnp.dot(h, pW2_ref[...], preferred_element_type=jnp.float32)
    spin_ref[...] = jnp.maximum(s + pb2_ref[...], 0.0)

    idx = idx_ref[0, 0, :]
    bsz = idx.shape[0]
    gids = jax.lax.broadcasted_iota(jnp.int32, (bsz, N_GRAPH), 1)
    onehot = (idx[:, None] == gids).astype(jnp.float32)
    ones_row = jnp.ones((1, bsz), jnp.float32)
    blk_counts = jnp.dot(ones_row, onehot,
                         preferred_element_type=jnp.float32)
    mask0 = (idx == 0).astype(jnp.float32)[None, :]
    blk_emb0 = jnp.dot(mask0, xb,
                       preferred_element_type=jnp.float32)

    @pl.when(i == 0)
    def _init():
        counts_ref[...] = blk_counts
        emb0_ref[...] = blk_emb0

    @pl.when(i != 0)
    def _acc():
        counts_ref[...] += blk_counts
        emb0_ref[...] += blk_emb0

    @pl.when(i == nb - 1)
    def _final():
        n = counts_ref[...]
        scale = jnp.sqrt(n).reshape(N_GRAPH, 1)
        z = emb0_ref[...] / scale
        h1 = jnp.dot(z, vW1_ref[...], preferred_element_type=jnp.float32)
        h1 = jnp.maximum(h1 + vb1_ref[...], 0.0)
        h2 = jnp.dot(h1, vW2_ref[...], preferred_element_type=jnp.float32)
        h2 = jnp.maximum(h2 + vb2_ref[...], 0.0)
        h3 = jnp.dot(h2, vW3_ref[...], preferred_element_type=jnp.float32)
        values_ref[...] = jnp.maximum(h3 + vb3_ref[...], 0.0)


@functools.partial(jax.jit, static_argnames=("interpret",))
def _run(x, idx, pW1, pb1, pW2, pb2, vW1, vb1, vW2, vb2, vW3, vb3,
         interpret=False):
    n_nodes, d_feat = x.shape
    block = 5000
    if n_nodes % block != 0:
        block = 8
        while n_nodes % (block * 2) == 0 and block < 4096:
            block *= 2
    nb = n_nodes // block

    idx3 = idx.reshape(nb, 1, block)

    in_specs = [
            pl.BlockSpec((block, d_feat), lambda i: (i, 0)),
            pl.BlockSpec((1, 1, block), lambda i: (i, 0, 0)),
            pl.BlockSpec(pW1.shape, lambda i: (0, 0)),
            pl.BlockSpec((1, pb1.shape[0]), lambda i: (0, 0)),
            pl.BlockSpec(pW2.shape, lambda i: (0, 0)),
            pl.BlockSpec((1, pb2.shape[0]), lambda i: (0, 0)),
            pl.BlockSpec(vW1.shape, lambda i: (0, 0)),
            pl.BlockSpec((1, vb1.shape[0]), lambda i: (0, 0)),
            pl.BlockSpec(vW2.shape, lambda i: (0, 0)),
            pl.BlockSpec((1, vb2.shape[0]), lambda i: (0, 0)),
            pl.BlockSpec(vW3.shape, lambda i: (0, 0)),
            pl.BlockSpec((1, vb3.shape[0]), lambda i: (0, 0)),
    ]
    out_specs = [
        pl.BlockSpec((block, 2), lambda i: (i, 0)),
        pl.BlockSpec((N_GRAPH, 1), lambda i: (0, 0)),
    ]

    spin, values = pl.pallas_call(
        _body,
        grid=(nb,),
        in_specs=in_specs,
        out_specs=out_specs,
        out_shape=[
            jax.ShapeDtypeStruct((n_nodes, 2), jnp.float32),
            jax.ShapeDtypeStruct((N_GRAPH, 1), jnp.float32),
        ],
        scratch_shapes=[
            pltpu.VMEM((1, N_GRAPH), jnp.float32),
            pltpu.VMEM((1, d_feat), jnp.float32),
        ],
        interpret=interpret,
    )(x, idx3, pW1, pb1.reshape(1, -1), pW2, pb2.reshape(1, -1),
      vW1, vb1.reshape(1, -1), vW2, vb2.reshape(1, -1),
      vW3, vb3.reshape(1, -1))
    return spin, values[:, 0]


def kernel(x, node_graph_idx, pW1, pb1, pW2, pb2, vW1, vb1, vW2, vb2, vW3, vb3):
    idx = node_graph_idx.astype(jnp.int32)
    return _run(x, idx, pW1, pb1, pW2, pb2, vW1, vb1, vW2, vb2, vW3, vb3)


# speedup vs baseline: 5.0334x; 1.1539x over previous
import functools

import jax
import jax.numpy as jnp
from jax.experimental import pallas as pl
from jax.experimental.pallas import tpu as pltpu

N_GRAPH = 64


def _body(x_ref, idx_ref, pW1_ref, pb1_ref, pW2_ref, pb2_ref,
          vW1_ref, vb1_ref, vW2_ref, vb2_ref, vW3_ref, vb3_ref,
          spin_ref, values_ref, counts_ref, emb0_ref):
    i = pl.program_id(0)
    nb = pl.num_programs(0)

    xb = x_ref[...]

    idx = idx_ref[0, 0, :]
    bsz = idx.shape[0]
    gids = jax.lax.broadcasted_iota(jnp.int32, (N_GRAPH, bsz), 0)
    onehot = (idx[None, :] == gids).astype(jnp.bfloat16)
    ones_col = jnp.ones((bsz, 1), jnp.bfloat16)
    blk_counts = jnp.dot(onehot, ones_col,
                         preferred_element_type=jnp.float32)
    mask0 = (idx == 0).astype(jnp.float32)[None, :]
    blk_emb0 = jnp.dot(mask0, xb,
                       preferred_element_type=jnp.float32)

    h = jnp.dot(xb, pW1_ref[...], preferred_element_type=jnp.float32)
    h = jnp.maximum(h + pb1_ref[...], 0.0)
    s = jnp.dot(h, pW2_ref[...], preferred_element_type=jnp.float32)
    spin_ref[...] = jnp.maximum(s + pb2_ref[...], 0.0)

    @pl.when(i == 0)
    def _init():
        counts_ref[...] = blk_counts
        emb0_ref[...] = blk_emb0

    @pl.when(i != 0)
    def _acc():
        counts_ref[...] += blk_counts
        emb0_ref[...] += blk_emb0

    @pl.when(i == nb - 1)
    def _final():
        scale = jnp.sqrt(counts_ref[...])
        z = emb0_ref[...] / scale
        h1 = jnp.dot(z, vW1_ref[...], preferred_element_type=jnp.float32)
        h1 = jnp.maximum(h1 + vb1_ref[...], 0.0)
        h2 = jnp.dot(h1, vW2_ref[...], preferred_element_type=jnp.float32)
        h2 = jnp.maximum(h2 + vb2_ref[...], 0.0)
        h3 = jnp.dot(h2, vW3_ref[...], preferred_element_type=jnp.float32)
        values_ref[...] = jnp.maximum(h3 + vb3_ref[...], 0.0)


@functools.partial(jax.jit, static_argnames=("interpret",))
def _run(x, idx, pW1, pb1, pW2, pb2, vW1, vb1, vW2, vb2, vW3, vb3,
         interpret=False):
    n_nodes, d_feat = x.shape
    block = 10000
    if n_nodes % block != 0:
        block = 8
        while n_nodes % (block * 2) == 0 and block < 4096:
            block *= 2
    nb = n_nodes // block

    idx3 = idx.reshape(nb, 1, block)

    in_specs = [
            pl.BlockSpec((block, d_feat), lambda i: (i, 0)),
            pl.BlockSpec((1, 1, block), lambda i: (i, 0, 0)),
            pl.BlockSpec(pW1.shape, lambda i: (0, 0)),
            pl.BlockSpec((1, pb1.shape[0]), lambda i: (0, 0)),
            pl.BlockSpec(pW2.shape, lambda i: (0, 0)),
            pl.BlockSpec((1, pb2.shape[0]), lambda i: (0, 0)),
            pl.BlockSpec(vW1.shape, lambda i: (0, 0)),
            pl.BlockSpec((1, vb1.shape[0]), lambda i: (0, 0)),
            pl.BlockSpec(vW2.shape, lambda i: (0, 0)),
            pl.BlockSpec((1, vb2.shape[0]), lambda i: (0, 0)),
            pl.BlockSpec(vW3.shape, lambda i: (0, 0)),
            pl.BlockSpec((1, vb3.shape[0]), lambda i: (0, 0)),
    ]
    out_specs = [
        pl.BlockSpec((block, 2), lambda i: (i, 0)),
        pl.BlockSpec((N_GRAPH, 1), lambda i: (0, 0)),
    ]

    spin, values = pl.pallas_call(
        _body,
        grid=(nb,),
        in_specs=in_specs,
        out_specs=out_specs,
        out_shape=[
            jax.ShapeDtypeStruct((n_nodes, 2), jnp.float32),
            jax.ShapeDtypeStruct((N_GRAPH, 1), jnp.float32),
        ],
        scratch_shapes=[
            pltpu.VMEM((N_GRAPH, 1), jnp.float32),
            pltpu.VMEM((1, d_feat), jnp.float32),
        ],
        interpret=interpret,
    )(x, idx3, pW1, pb1.reshape(1, -1), pW2, pb2.reshape(1, -1),
      vW1, vb1.reshape(1, -1), vW2, vb2.reshape(1, -1),
      vW3, vb3.reshape(1, -1))
    return spin, values[:, 0]


def kernel(x, node_graph_idx, pW1, pb1, pW2, pb2, vW1, vb1, vW2, vb2, vW3, vb3):
    idx = node_graph_idx.astype(jnp.int32)
    return _run(x, idx, pW1, pb1, pW2, pb2, vW1, vb1, vW2, vb2, vW3, vb3)
